# baseline (speedup 1.0000x reference)
import functools

import jax
import jax.numpy as jnp
from jax import lax
from jax.experimental import pallas as pl
from jax.experimental.pallas import tpu as pltpu
from jax.experimental.pallas import tpu_sc as plsc

_VOCAB = 100000
_BATCH = 16384
_DIM = 64
_NW = 32
_BPW = _BATCH // _NW
_CHUNK = 128
_NCHUNK = _BPW // _CHUNK
_LANES = 16
_TBL = _VOCAB * _DIM


def _cbow_body(xf_hbm, tab_hbm, out_hbm, xbuf, rows0, rows1, rows2, stage,
               out_v, sem):
    wid = lax.axis_index("s") * 2 + lax.axis_index("c")
    base = wid * _BPW

    pltpu.sync_copy(
        xf_hbm.at[pl.ds(base * 3, _BPW * 3)], xbuf.at[pl.ds(0, _BPW * 3)]
    )

    def fire_row(k, _):
        v = xbuf[pl.ds(3 * k, _LANES)]
        i0 = v[0]
        i1 = v[1]
        i2 = v[2]
        pltpu.async_copy(
            tab_hbm.at[pl.ds(i0 * 128, _DIM)],
            rows0.at[pl.ds(k * _DIM, _DIM)], sem)
        pltpu.async_copy(
            tab_hbm.at[pl.ds(i1 * 128, _DIM)],
            rows1.at[pl.ds(k * _DIM, _DIM)], sem)
        pltpu.async_copy(
            tab_hbm.at[pl.ds(i2 * 128 + _DIM, _DIM)],
            rows2.at[pl.ds(k * _DIM, _DIM)], sem)
        return 0

    def fire_chunk(j):
        lax.fori_loop(j * _CHUNK, (j + 1) * _CHUNK, fire_row, 0)

    def drain_chunk(j):
        n = _CHUNK * _DIM
        for rows in (rows0, rows1, rows2):
            pltpu.make_async_copy(
                tab_hbm.at[pl.ds(0, n)], rows.at[pl.ds(j * n, n)], sem
            ).wait()

    lane17 = lax.iota(jnp.int32, _LANES) * 17

    def group_body(g, _):
        r = g * _LANES
        for l in range(_LANES):
            vacc = jnp.zeros((_LANES,), jnp.float32)
            for c in range(_DIM // _LANES):
                off = (r + l) * _DIM + c * _LANES
                a0 = rows0[pl.ds(off, _LANES)]
                a1 = rows1[pl.ds(off, _LANES)]
                bv = rows2[pl.ds(off, _LANES)]
                vacc = vacc + (a0 + a1) * bv
            stage[pl.ds(l * 17, _LANES)] = vacc
        acc = jnp.zeros((_LANES,), jnp.float32)
        for c in range(_LANES):
            acc = acc + plsc.load_gather(stage, [lane17 + c])
        out_v[pl.ds(r, _LANES)] = acc * 0.5
        return 0

    groups_per_chunk = _CHUNK // _LANES
    fire_chunk(0)
    for j in range(_NCHUNK):
        drain_chunk(j)
        if j + 1 < _NCHUNK:
            fire_chunk(j + 1)
        lax.fori_loop(
            j * groups_per_chunk, (j + 1) * groups_per_chunk, group_body, 0
        )

    pltpu.sync_copy(out_v, out_hbm.at[pl.ds(base, _BPW)])


def _pack_body(a_ref, b_ref, o_ref):
    o_ref[:, 0:_DIM] = a_ref[...]
    o_ref[:, _DIM:128] = b_ref[...]


@jax.jit
def _cbow(xf, A, B):
    packed = pl.pallas_call(
        _pack_body,
        out_shape=jax.ShapeDtypeStruct((_VOCAB, 128), jnp.float32),
        grid=(50,),
        in_specs=[
            pl.BlockSpec((_VOCAB // 50, _DIM), lambda i: (i, 0)),
            pl.BlockSpec((_VOCAB // 50, _DIM), lambda i: (i, 0)),
        ],
        out_specs=pl.BlockSpec((_VOCAB // 50, 128), lambda i: (i, 0)),
    )(A, B)
    tab = packed.reshape(-1)
    mesh = plsc.VectorSubcoreMesh(core_axis_name="c", subcore_axis_name="s")
    f = pl.kernel(
        _cbow_body,
        out_type=jax.ShapeDtypeStruct((_BATCH,), jnp.float32),
        mesh=mesh,
        scratch_types=[
            pltpu.VMEM((_BPW * 3 + _LANES,), jnp.int32),
            pltpu.VMEM((_BPW * _DIM,), jnp.float32),
            pltpu.VMEM((_BPW * _DIM,), jnp.float32),
            pltpu.VMEM((_BPW * _DIM,), jnp.float32),
            pltpu.VMEM((_LANES * 17,), jnp.float32),
            pltpu.VMEM((_BPW,), jnp.float32),
            pltpu.SemaphoreType.DMA,
        ],
        compiler_params=pltpu.CompilerParams(
            needs_layout_passes=False, use_tc_tiling_on_sc=False
        ),
    )
    return f(xf, tab)


def kernel(x, A, B):
    xf = jnp.abs(x.astype(jnp.int32).reshape(-1))
    return _cbow(xf, A, B)


# speedup vs baseline: 1.3428x; 1.3428x over previous
import functools

import jax
import jax.numpy as jnp
from jax import lax
from jax.experimental import pallas as pl
from jax.experimental.pallas import tpu as pltpu
from jax.experimental.pallas import tpu_sc as plsc

_VOCAB = 100000
_BATCH = 16384
_DIM = 64
_NW = 32
_BPW = _BATCH // _NW
_CHUNK = 128
_NCHUNK = _BPW // _CHUNK
_LANES = 16
_TBL = _VOCAB * _DIM


def _cbow_body(xf_hbm, tab_hbm, out_hbm, xbuf, rows0, rows1, rows2, stage,
               out_v, sem):
    wid = lax.axis_index("s") * 2 + lax.axis_index("c")
    base = wid * _BPW

    pltpu.sync_copy(
        xf_hbm.at[pl.ds(base * 3, _BPW * 3)], xbuf.at[pl.ds(0, _BPW * 3)]
    )

    def fire_row(k, _):
        v = xbuf[pl.ds(3 * k, _LANES)]
        i0 = v[0]
        i1 = v[1]
        i2 = v[2]
        pltpu.async_copy(
            tab_hbm.at[pl.ds(i0 * 128, _DIM)],
            rows0.at[pl.ds(k * _DIM, _DIM)], sem)
        pltpu.async_copy(
            tab_hbm.at[pl.ds(i1 * 128, _DIM)],
            rows1.at[pl.ds(k * _DIM, _DIM)], sem)
        pltpu.async_copy(
            tab_hbm.at[pl.ds(i2 * 128 + _DIM, _DIM)],
            rows2.at[pl.ds(k * _DIM, _DIM)], sem)
        return 0

    def fire_chunk(j):
        lax.fori_loop(j * _CHUNK, (j + 1) * _CHUNK, fire_row, 0)

    def drain_chunk(j):
        n = _CHUNK * _DIM
        for rows in (rows0, rows1, rows2):
            pltpu.make_async_copy(
                tab_hbm.at[pl.ds(0, n)], rows.at[pl.ds(j * n, n)], sem
            ).wait()

    lane17 = lax.iota(jnp.int32, _LANES) * 17

    def group_body(g, _):
        r = g * _LANES
        for l in range(_LANES):
            vacc = jnp.zeros((_LANES,), jnp.float32)
            for c in range(_DIM // _LANES):
                off = (r + l) * _DIM + c * _LANES
                a0 = rows0[pl.ds(off, _LANES)]
                a1 = rows1[pl.ds(off, _LANES)]
                bv = rows2[pl.ds(off, _LANES)]
                vacc = vacc + (a0 + a1) * bv
            stage[pl.ds(l * 17, _LANES)] = vacc
        acc = jnp.zeros((_LANES,), jnp.float32)
        for c in range(_LANES):
            acc = acc + plsc.load_gather(stage, [lane17 + c])
        out_v[pl.ds(r, _LANES)] = acc * 0.5
        return 0

    groups_per_chunk = _CHUNK // _LANES
    fire_chunk(0)
    for j in range(_NCHUNK):
        drain_chunk(j)
        if j + 1 < _NCHUNK:
            fire_chunk(j + 1)
        lax.fori_loop(
            j * groups_per_chunk, (j + 1) * groups_per_chunk, group_body, 0
        )

    pltpu.sync_copy(out_v, out_hbm.at[pl.ds(base, _BPW)])


@jax.jit
def _cbow(xf, A, B):
    tab = jnp.concatenate([A, B], axis=1).reshape(-1)
    mesh = plsc.VectorSubcoreMesh(core_axis_name="c", subcore_axis_name="s")
    f = pl.kernel(
        _cbow_body,
        out_type=jax.ShapeDtypeStruct((_BATCH,), jnp.float32),
        mesh=mesh,
        scratch_types=[
            pltpu.VMEM((_BPW * 3 + _LANES,), jnp.int32),
            pltpu.VMEM((_BPW * _DIM,), jnp.float32),
            pltpu.VMEM((_BPW * _DIM,), jnp.float32),
            pltpu.VMEM((_BPW * _DIM,), jnp.float32),
            pltpu.VMEM((_LANES * 17,), jnp.float32),
            pltpu.VMEM((_BPW,), jnp.float32),
            pltpu.SemaphoreType.DMA,
        ],
        compiler_params=pltpu.CompilerParams(
            needs_layout_passes=False, use_tc_tiling_on_sc=False
        ),
    )
    return f(xf, tab)


def kernel(x, A, B):
    xf = jnp.abs(x.astype(jnp.int32).reshape(-1))
    return _cbow(xf, A, B)
